# Initial kernel scaffold; baseline (speedup 1.0000x reference)
import jax
import jax.numpy as jnp
from jax import lax
from jax.experimental import pallas as pl
from jax.experimental.pallas import tpu as pltpu
from jax.experimental.pallas import tpu_sc as plsc

N = 10000
E = 160000
F_IN = 128
F_OUT = 600
PERIODS = 4

CHUNK = 128
E_PAD = 163840
NCHUNKS = E_PAD // CHUNK
NC, NS = 2, 16
BN = 1000

_MESH = plsc.VectorSubcoreMesh(core_axis_name="c", subcore_axis_name="s")
_F32 = jnp.float32


def _sc_deg_body(colp, ewp, z1d, degp, idxb, payb, acc):
    c = lax.axis_index("c")
    s = lax.axis_index("s")

    @pl.when(s < 15)
    def _():
        pltpu.sync_copy(z1d, acc.at[pl.ds(s * 640, 640)])

    @pl.when(s == 15)
    def _():
        pltpu.sync_copy(z1d.at[pl.ds(0, 400)], acc.at[pl.ds(9600, 400)])

    plsc.subcore_barrier()
    w = s * NC + c

    @pl.loop(0, NCHUNKS // (NC * NS))
    def _(j):
        k = w * (NCHUNKS // (NC * NS)) + j
        pltpu.sync_copy(colp.at[pl.ds(k, 1)], idxb)
        pltpu.sync_copy(ewp.at[pl.ds(k, 1)], payb)
        pltpu.sync_copy(payb.at[0], acc.at[idxb.at[0]], add=True)

    plsc.subcore_barrier()

    @pl.when(s < 15)
    def _():
        pltpu.sync_copy(acc.at[pl.ds(s * 640, 640)],
                        degp.at[c, pl.ds(s * 640, 640)])

    @pl.when(s == 15)
    def _():
        pltpu.sync_copy(acc.at[pl.ds(9600, 400)],
                        degp.at[c, pl.ds(9600, 400)])


_sc_deg = pl.kernel(
    _sc_deg_body,
    out_type=jax.ShapeDtypeStruct((NC, N), _F32),
    mesh=_MESH,
    scratch_types=[
        pltpu.VMEM((1, CHUNK), jnp.int32),
        pltpu.VMEM((1, CHUNK), _F32),
        pltpu.VMEM_SHARED((N,), _F32),
    ],
)


def _sc_agg_body(rowp, colp, ewp, z2d, y0, y1, y2, y3,
                 s0, s1, s2, s3, rowb, colb, ewb, gbuf, acc):
    c = lax.axis_index("c")
    s = lax.axis_index("s")

    def period(y, out):
        @pl.when(s < 15)
        def _():
            pltpu.sync_copy(z2d, acc.at[pl.ds(s * 640, 640)])

        @pl.when(s == 15)
        def _():
            pltpu.sync_copy(z2d.at[pl.ds(0, 400)], acc.at[pl.ds(9600, 400)])

        plsc.subcore_barrier()

        @pl.loop(0, NCHUNKS // NS)
        def _(j):
            k = s * (NCHUNKS // NS) + j
            pltpu.sync_copy(rowp.at[pl.ds(k, 1)], rowb)
            pltpu.sync_copy(colp.at[pl.ds(k, 1)], colb)
            pltpu.sync_copy(ewp.at[pl.ds(k, 1)], ewb)
            pltpu.sync_copy(y.at[rowb.at[0]], gbuf)

            @pl.loop(0, CHUNK)
            def _(e):
                sc = ewb[0, e]
                for q in range(F_IN // 16):
                    sl = pl.ds(q * 16, 16)
                    gbuf[e, sl] = gbuf[e, sl] * sc

            pltpu.sync_copy(gbuf, acc.at[colb.at[0]], add=True)

        plsc.subcore_barrier()

        @pl.when(s < 15)
        def _():
            pltpu.sync_copy(acc.at[pl.ds(s * 640, 640)],
                            out.at[pl.ds(s * 640, 640)])

        @pl.when(s == 15)
        def _():
            pltpu.sync_copy(acc.at[pl.ds(9600, 400)],
                            out.at[pl.ds(9600, 400)])

        plsc.subcore_barrier()

    @pl.when(c == 0)
    def _():
        period(y0, s0)
        period(y1, s1)

    @pl.when(c == 1)
    def _():
        period(y2, s2)
        period(y3, s3)


_sc_agg = pl.kernel(
    _sc_agg_body,
    out_type=[jax.ShapeDtypeStruct((N, F_IN), _F32)] * PERIODS,
    mesh=_MESH,
    scratch_types=[
        pltpu.VMEM((1, CHUNK), jnp.int32),
        pltpu.VMEM((1, CHUNK), jnp.int32),
        pltpu.VMEM((1, CHUNK), _F32),
        pltpu.VMEM((CHUNK, F_IN), _F32),
        pltpu.VMEM_SHARED((N, F_IN), _F32),
    ],
)


def _tc_pre_body(degp, xt, dinv, y0, y1, y2, y3):
    d = degp[0, :] + degp[1, :] + 1.0
    dv = lax.rsqrt(d)
    dinv[...] = dv
    dvc = dv[:, None]
    yrefs = (y0, y1, y2, y3)
    for t in range(PERIODS):
        yrefs[t][...] = xt[t] * dvc


def _tc_pre(degp, xt):
    return pl.pallas_call(
        _tc_pre_body,
        grid=(N // BN,),
        in_specs=[
            pl.BlockSpec((NC, BN), lambda i: (0, i)),
            pl.BlockSpec((PERIODS, BN, F_IN), lambda i: (0, i, 0)),
        ],
        out_specs=[pl.BlockSpec((BN,), lambda i: (i,))]
        + [pl.BlockSpec((BN, F_IN), lambda i: (i, 0))] * PERIODS,
        out_shape=[jax.ShapeDtypeStruct((N,), _F32)]
        + [jax.ShapeDtypeStruct((N, F_IN), _F32)] * PERIODS,
    )(degp, xt)


def _tc_fold_body(wz, bz, wlz, blz, wh, bh, wlh, blh, az, azb, ah, ahb):
    hi = jax.lax.Precision.HIGHEST
    wlza = wlz[0:F_OUT, :]
    wlha = wlh[0:F_OUT, :]
    az[...] = jnp.dot(wz[...], wlza, precision=hi)
    azb[...] = jnp.dot(bz[...][None, :], wlza, precision=hi)[0] + blz[...]
    ah[...] = jnp.dot(wh[...], wlha, precision=hi)
    ahb[...] = jnp.dot(bh[...][None, :], wlha, precision=hi)[0] + blh[...]


def _tc_fold(wz, bz, wlz, blz, wh, bh, wlh, blh):
    return pl.pallas_call(
        _tc_fold_body,
        out_shape=[
            jax.ShapeDtypeStruct((F_IN, F_OUT), _F32),
            jax.ShapeDtypeStruct((F_OUT,), _F32),
            jax.ShapeDtypeStruct((F_IN, F_OUT), _F32),
            jax.ShapeDtypeStruct((F_OUT,), _F32),
        ],
    )(wz, bz, wlz, blz, wh, bh, wlh, blh)


def _tc_dense_body(att, az, azb, ah, ahb, wo, bo, dinv,
                   y0, y1, y2, y3, s0, s1, s2, s3, out):
    hi = jax.lax.Precision.HIGHEST
    a = att[...]
    e = jnp.exp(a - jnp.max(a))
    probs = e / jnp.sum(e)
    dv = dinv[...][:, None]
    azm, ahm = az[...], ah[...]
    azv, ahv = azb[...][None, :], ahb[...][None, :]
    yr = (y0, y1, y2, y3)
    sr = (s0, s1, s2, s3)
    acc = jnp.zeros((BN, F_OUT), _F32)
    for t in range(PERIODS):
        p = dv * (sr[t][...] + yr[t][...])
        z = jax.nn.sigmoid(jnp.dot(p, azm, precision=hi) + azv)
        ht = jnp.tanh(jnp.dot(p, ahm, precision=hi) + ahv)
        acc = acc + probs[t] * ((1.0 - z) * ht)
    out[...] = jnp.dot(jax.nn.relu(acc), wo[...], precision=hi) + bo[...][None, :]


def _tc_dense(att, az, azb, ah, ahb, wo, bo, dinv, ys, ss):
    nf = pl.BlockSpec((BN, F_IN), lambda i: (i, 0))
    return pl.pallas_call(
        _tc_dense_body,
        grid=(N // BN,),
        in_specs=[
            pl.BlockSpec((PERIODS,), lambda i: (0,)),
            pl.BlockSpec((F_IN, F_OUT), lambda i: (0, 0)),
            pl.BlockSpec((F_OUT,), lambda i: (0,)),
            pl.BlockSpec((F_IN, F_OUT), lambda i: (0, 0)),
            pl.BlockSpec((F_OUT,), lambda i: (0,)),
            pl.BlockSpec((F_OUT, PERIODS), lambda i: (0, 0)),
            pl.BlockSpec((PERIODS,), lambda i: (0,)),
            pl.BlockSpec((BN,), lambda i: (i,)),
        ] + [nf] * (2 * PERIODS),
        out_specs=pl.BlockSpec((BN, PERIODS), lambda i: (i, 0)),
        out_shape=jax.ShapeDtypeStruct((N, PERIODS), _F32),
    )(att, az, azb, ah, ahb, wo, bo, dinv, *ys, *ss)


def kernel(x, edge_index, edge_attributes, attention, Wz, bz, Wlz, blz,
           Wr, br, Wlr, blr, Wh, bh, Wlh, blh, Wo, bo):
    pad = E_PAD - E
    row = jnp.concatenate([edge_index[0], jnp.zeros((pad,), edge_index.dtype)])
    col = jnp.concatenate([edge_index[1], jnp.zeros((pad,), edge_index.dtype)])
    ew = jnp.concatenate([edge_attributes, jnp.zeros((pad,), _F32)])
    rowp = row.reshape(NCHUNKS, CHUNK)
    colp = col.reshape(NCHUNKS, CHUNK)
    ewp = ew.reshape(NCHUNKS, CHUNK)
    xt = jnp.transpose(x, (2, 0, 1))
    z1d = jnp.zeros((640,), _F32)
    z2d = jnp.zeros((640, F_IN), _F32)

    degp = _sc_deg(colp, ewp, z1d)
    dinv, *ys = _tc_pre(degp, xt)
    az, azb, ah, ahb = _tc_fold(Wz, bz, Wlz, blz, Wh, bh, Wlh, blh)
    ss = _sc_agg(rowp, colp, ewp, z2d, *ys)
    return _tc_dense(attention, az, azb, ah, ahb, Wo, bo, dinv, ys, ss)


# speedup vs baseline: 33.5807x; 33.5807x over previous
import jax
import jax.numpy as jnp
from jax import lax
from jax.experimental import pallas as pl
from jax.experimental.pallas import tpu as pltpu
from jax.experimental.pallas import tpu_sc as plsc

N = 10000
E = 160000
F_IN = 128
F_OUT = 600
PERIODS = 4

CHUNK = 128
E_PAD = 163840
NCHUNKS = E_PAD // CHUNK
NC, NS = 2, 16
BN = 1000

_MESH = plsc.VectorSubcoreMesh(core_axis_name="c", subcore_axis_name="s")
_F32 = jnp.float32


N_PAD = 10240


def _sc_deg_body(colp, ewp, z1d, degp, idxb, payb, acc):
    c = lax.axis_index("c")
    s = lax.axis_index("s")
    pltpu.sync_copy(z1d, acc.at[pl.ds(s * 640, 640)])
    plsc.subcore_barrier()
    w = s * NC + c

    @pl.loop(0, NCHUNKS // (NC * NS))
    def _(j):
        k = w * (NCHUNKS // (NC * NS)) + j
        pltpu.sync_copy(colp.at[pl.ds(k, 1)], idxb)
        pltpu.sync_copy(ewp.at[pl.ds(k, 1)], payb)
        pltpu.sync_copy(payb.at[0], acc.at[idxb.at[0]], add=True)

    plsc.subcore_barrier()
    pltpu.sync_copy(acc.at[pl.ds(s * 640, 640)], degp.at[c, s])


_sc_deg = pl.kernel(
    _sc_deg_body,
    out_type=jax.ShapeDtypeStruct((NC, NS, 640), _F32),
    mesh=_MESH,
    scratch_types=[
        pltpu.VMEM((1, CHUNK), jnp.int32),
        pltpu.VMEM((1, CHUNK), _F32),
        pltpu.VMEM_SHARED((N_PAD,), _F32),
    ],
)


def _sc_agg_body(rowp, colp, ewp, z2d, y0, y1, y2, y3,
                 s0, s1, s2, s3, rowb, colb, ewb, gbuf, acc):
    c = lax.axis_index("c")
    s = lax.axis_index("s")

    def period(y, out):
        pltpu.sync_copy(z2d, acc.at[pl.ds(s * 640, 640)])
        plsc.subcore_barrier()

        @pl.loop(0, NCHUNKS // NS)
        def _(j):
            k = s * (NCHUNKS // NS) + j
            pltpu.sync_copy(rowp.at[pl.ds(k, 1)], rowb)
            pltpu.sync_copy(colp.at[pl.ds(k, 1)], colb)
            pltpu.sync_copy(ewp.at[pl.ds(k, 1)], ewb)
            pltpu.sync_copy(y.at[rowb.at[0]], gbuf)

            @pl.loop(0, CHUNK // 16)
            def _(g):
                ve = ewb[0, pl.ds(g * 16, 16)]
                for l in range(16):
                    sc = ve[l]
                    e = g * 16 + l
                    for q in range(F_IN // 16):
                        sl = pl.ds(q * 16, 16)
                        gbuf[e, sl] = gbuf[e, sl] * sc

            pltpu.sync_copy(gbuf, acc.at[colb.at[0]], add=True)

        plsc.subcore_barrier()
        pltpu.sync_copy(acc.at[pl.ds(s * 640, 640)], out.at[s])
        plsc.subcore_barrier()

    @pl.when(c == 0)
    def _():
        period(y0, s0)
        period(y1, s1)

    @pl.when(c == 1)
    def _():
        period(y2, s2)
        period(y3, s3)


_sc_agg = pl.kernel(
    _sc_agg_body,
    out_type=[jax.ShapeDtypeStruct((NS, 640, F_IN), _F32)] * PERIODS,
    mesh=_MESH,
    scratch_types=[
        pltpu.VMEM((1, CHUNK), jnp.int32),
        pltpu.VMEM((1, CHUNK), jnp.int32),
        pltpu.VMEM((1, CHUNK), _F32),
        pltpu.VMEM((CHUNK, F_IN), _F32),
        pltpu.VMEM_SHARED((N_PAD, F_IN), _F32),
    ],
)


def _tc_pre_body(degp, xt, dinv, y0, y1, y2, y3):
    d = degp[:, 0] + degp[:, 1] + 1.0
    dvc = lax.rsqrt(d)[:, None]
    dinv[...] = dvc
    yrefs = (y0, y1, y2, y3)
    for t in range(PERIODS):
        yrefs[t][...] = xt[t] * dvc


def _tc_pre(degp, xt):
    return pl.pallas_call(
        _tc_pre_body,
        grid=(N // BN,),
        in_specs=[
            pl.BlockSpec((BN, NC), lambda i: (i, 0)),
            pl.BlockSpec((PERIODS, BN, F_IN), lambda i: (0, i, 0)),
        ],
        out_specs=[pl.BlockSpec((BN, 1), lambda i: (i, 0))]
        + [pl.BlockSpec((BN, F_IN), lambda i: (i, 0))] * PERIODS,
        out_shape=[jax.ShapeDtypeStruct((N, 1), _F32)]
        + [jax.ShapeDtypeStruct((N, F_IN), _F32)] * PERIODS,
    )(degp, xt)


def _tc_fold_body(wz, bz, wlz, blz, wh, bh, wlh, blh, az, azb, ah, ahb):
    hi = jax.lax.Precision.HIGHEST
    wlza = wlz[0:F_OUT, :]
    wlha = wlh[0:F_OUT, :]
    az[...] = jnp.dot(wz[...], wlza, precision=hi)
    azb[...] = jnp.dot(bz[...][None, :], wlza, precision=hi)[0] + blz[...]
    ah[...] = jnp.dot(wh[...], wlha, precision=hi)
    ahb[...] = jnp.dot(bh[...][None, :], wlha, precision=hi)[0] + blh[...]


def _tc_fold(wz, bz, wlz, blz, wh, bh, wlh, blh):
    return pl.pallas_call(
        _tc_fold_body,
        out_shape=[
            jax.ShapeDtypeStruct((F_IN, F_OUT), _F32),
            jax.ShapeDtypeStruct((F_OUT,), _F32),
            jax.ShapeDtypeStruct((F_IN, F_OUT), _F32),
            jax.ShapeDtypeStruct((F_OUT,), _F32),
        ],
    )(wz, bz, wlz, blz, wh, bh, wlh, blh)


def _tc_dense_body(att, az, azb, ah, ahb, wo, bo, dinv,
                   y0, y1, y2, y3, s0, s1, s2, s3, out):
    hi = jax.lax.Precision.HIGHEST
    a = att[...]
    e = jnp.exp(a - jnp.max(a))
    probs = e / jnp.sum(e)
    dv = dinv[...]
    azm, ahm = az[...], ah[...]
    azv, ahv = azb[...][None, :], ahb[...][None, :]
    yr = (y0, y1, y2, y3)
    sr = (s0, s1, s2, s3)
    acc = jnp.zeros((BN, F_OUT), _F32)
    for t in range(PERIODS):
        p = dv * (sr[t][...] + yr[t][...])
        z = jax.nn.sigmoid(jnp.dot(p, azm, precision=hi) + azv)
        ht = jnp.tanh(jnp.dot(p, ahm, precision=hi) + ahv)
        acc = acc + probs[t] * ((1.0 - z) * ht)
    out[...] = jnp.dot(jax.nn.relu(acc), wo[...], precision=hi) + bo[...][None, :]


def _tc_dense(att, az, azb, ah, ahb, wo, bo, dinv, ys, ss):
    nf = pl.BlockSpec((BN, F_IN), lambda i: (i, 0))
    return pl.pallas_call(
        _tc_dense_body,
        grid=(N // BN,),
        in_specs=[
            pl.BlockSpec((PERIODS,), lambda i: (0,)),
            pl.BlockSpec((F_IN, F_OUT), lambda i: (0, 0)),
            pl.BlockSpec((F_OUT,), lambda i: (0,)),
            pl.BlockSpec((F_IN, F_OUT), lambda i: (0, 0)),
            pl.BlockSpec((F_OUT,), lambda i: (0,)),
            pl.BlockSpec((F_OUT, PERIODS), lambda i: (0, 0)),
            pl.BlockSpec((PERIODS,), lambda i: (0,)),
            pl.BlockSpec((BN, 1), lambda i: (i, 0)),
        ] + [nf] * (2 * PERIODS),
        out_specs=pl.BlockSpec((BN, PERIODS), lambda i: (i, 0)),
        out_shape=jax.ShapeDtypeStruct((N, PERIODS), _F32),
    )(att, az, azb, ah, ahb, wo, bo, dinv, *ys, *ss)


def kernel(x, edge_index, edge_attributes, attention, Wz, bz, Wlz, blz,
           Wr, br, Wlr, blr, Wh, bh, Wlh, blh, Wo, bo):
    pad = E_PAD - E
    row = jnp.concatenate([edge_index[0], jnp.zeros((pad,), edge_index.dtype)])
    col = jnp.concatenate([edge_index[1], jnp.zeros((pad,), edge_index.dtype)])
    ew = jnp.concatenate([edge_attributes, jnp.zeros((pad,), _F32)])
    rowp = row.reshape(NCHUNKS, CHUNK)
    colp = col.reshape(NCHUNKS, CHUNK)
    ewp = ew.reshape(NCHUNKS, CHUNK)
    xt = jnp.transpose(x, (2, 0, 1))
    z1d = jnp.zeros((640,), _F32)
    z2d = jnp.zeros((640, F_IN), _F32)

    degp = _sc_deg(colp, ewp, z1d)
    degp2 = degp.reshape(NC, N_PAD)[:, :N].T
    dinv, *ys = _tc_pre(degp2, xt)
    az, azb, ah, ahb = _tc_fold(Wz, bz, Wlz, blz, Wh, bh, Wlh, blh)
    ss = _sc_agg(rowp, colp, ewp, z2d, *ys)
    ss = [s.reshape(N_PAD, F_IN)[:N] for s in ss]
    return _tc_dense(attention, az, azb, ah, ahb, Wo, bo, dinv, ys, ss)


# speedup vs baseline: 46.4074x; 1.3820x over previous
import jax
import jax.numpy as jnp
from jax import lax
from jax.experimental import pallas as pl
from jax.experimental.pallas import tpu as pltpu
from jax.experimental.pallas import tpu_sc as plsc

N = 10000
E = 160000
F_IN = 128
F_OUT = 600
PERIODS = 4

CHUNK = 128
E_PAD = 163840
NCHUNKS = E_PAD // CHUNK
NC, NS = 2, 16
BN = 1000

_MESH = plsc.VectorSubcoreMesh(core_axis_name="c", subcore_axis_name="s")
_F32 = jnp.float32


N_PAD = 10240


def _sc_deg_body(colp, ewp, z1d, degp, idxb, payb, acc):
    c = lax.axis_index("c")
    s = lax.axis_index("s")
    pltpu.sync_copy(z1d, acc.at[pl.ds(s * 640, 640)])
    plsc.subcore_barrier()
    w = s * NC + c

    @pl.loop(0, NCHUNKS // (NC * NS))
    def _(j):
        k = w * (NCHUNKS // (NC * NS)) + j
        pltpu.sync_copy(colp.at[pl.ds(k, 1)], idxb)
        pltpu.sync_copy(ewp.at[pl.ds(k, 1)], payb)
        pltpu.sync_copy(payb.at[0], acc.at[idxb.at[0]], add=True)

    plsc.subcore_barrier()
    pltpu.sync_copy(acc.at[pl.ds(s * 640, 640)], degp.at[c, s])


_sc_deg = pl.kernel(
    _sc_deg_body,
    out_type=jax.ShapeDtypeStruct((NC, NS, 640), _F32),
    mesh=_MESH,
    scratch_types=[
        pltpu.VMEM((1, CHUNK), jnp.int32),
        pltpu.VMEM((1, CHUNK), _F32),
        pltpu.VMEM_SHARED((N_PAD,), _F32),
    ],
)


NPT = NCHUNKS // NS
NBUF = 4


def _sc_agg_body(rowp, colp, ewp, z2d, y0, y1, y2, y3,
                 s0, s1, s2, s3, rcb, ewb, gbuf, acc, sem_e, sem_g, sem_s):
    c = lax.axis_index("c")
    s = lax.axis_index("s")

    def period(y, out):
        pltpu.sync_copy(z2d, acc.at[pl.ds(s * 640, 640)])
        plsc.subcore_barrier()
        base = s * NPT

        def gslot(u):
            return gbuf.at[pl.ds((u % 2) * CHUNK, CHUNK)]

        def e_issue(k, u):
            kk = pl.ds(base + k, 1)
            pltpu.async_copy(rowp.at[kk], rcb.at[pl.ds(2 * u, 1)], sem_e)
            pltpu.async_copy(colp.at[kk], rcb.at[pl.ds(2 * u + 1, 1)], sem_e)
            pltpu.async_copy(ewp.at[kk], ewb.at[pl.ds(u, 1)], sem_e)

        def e_wait(u):
            k0 = pl.ds(0, 1)
            pltpu.make_async_copy(rowp.at[k0], rcb.at[pl.ds(2 * u, 1)],
                                  sem_e).wait()
            pltpu.make_async_copy(colp.at[k0], rcb.at[pl.ds(2 * u + 1, 1)],
                                  sem_e).wait()
            pltpu.make_async_copy(ewp.at[k0], ewb.at[pl.ds(u, 1)],
                                  sem_e).wait()

        def g_issue(u):
            pltpu.async_copy(y.at[rcb.at[2 * u]], gslot(u), sem_g)

        def g_wait(u):
            pltpu.make_async_copy(y.at[pl.ds(0, CHUNK)], gslot(u),
                                  sem_g).wait()

        def sc_issue(u):
            pltpu.async_copy(gslot(u), acc.at[rcb.at[2 * u + 1]], sem_s,
                             add=True)

        def sc_wait(u):
            pltpu.make_async_copy(y.at[pl.ds(0, CHUNK)], gslot(u),
                                  sem_s).wait()

        e_issue(0, 0)
        e_issue(1, 1)
        e_wait(0)
        g_issue(0)

        @pl.loop(0, NPT // NBUF)
        def _(jj):
            for u in range(NBUF):
                j = jj * NBUF + u
                if u == 0:
                    @pl.when(jj > 0)
                    def _():
                        sc_wait(NBUF - 1)
                else:
                    sc_wait(u - 1)

                @pl.when(j + 2 < NPT)
                def _():
                    e_issue(j + 2, (u + 2) % NBUF)

                @pl.when(j + 1 < NPT)
                def _():
                    e_wait((u + 1) % NBUF)
                    g_issue((u + 1) % NBUF)

                g_wait(u)

                @pl.loop(0, CHUNK // 16)
                def _(g):
                    ve = ewb[u, pl.ds(g * 16, 16)]
                    for l in range(16):
                        scl = ve[l]
                        e = (u % 2) * CHUNK + g * 16 + l
                        for q in range(F_IN // 16):
                            slq = pl.ds(q * 16, 16)
                            gbuf[e, slq] = gbuf[e, slq] * scl

                sc_issue(u)

        sc_wait(NBUF - 1)
        plsc.subcore_barrier()
        pltpu.sync_copy(acc.at[pl.ds(s * 640, 640)], out.at[s])
        plsc.subcore_barrier()

    @pl.when(c == 0)
    def _():
        period(y0, s0)
        period(y1, s1)

    @pl.when(c == 1)
    def _():
        period(y2, s2)
        period(y3, s3)


_sc_agg = pl.kernel(
    _sc_agg_body,
    out_type=[jax.ShapeDtypeStruct((NS, 640, F_IN), _F32)] * PERIODS,
    mesh=_MESH,
    scratch_types=[
        pltpu.VMEM((NBUF * 2, CHUNK), jnp.int32),
        pltpu.VMEM((NBUF, CHUNK), _F32),
        pltpu.VMEM((2 * CHUNK, F_IN), _F32),
        pltpu.VMEM_SHARED((N_PAD, F_IN), _F32),
        pltpu.SemaphoreType.DMA,
        pltpu.SemaphoreType.DMA,
        pltpu.SemaphoreType.DMA,
    ],
)


def _tc_pre_body(degp, xt, dinv, y0, y1, y2, y3):
    d = degp[:, 0] + degp[:, 1] + 1.0
    dvc = lax.rsqrt(d)[:, None]
    dinv[...] = dvc
    yrefs = (y0, y1, y2, y3)
    for t in range(PERIODS):
        yrefs[t][...] = xt[t] * dvc


def _tc_pre(degp, xt):
    return pl.pallas_call(
        _tc_pre_body,
        grid=(N // BN,),
        in_specs=[
            pl.BlockSpec((BN, NC), lambda i: (i, 0)),
            pl.BlockSpec((PERIODS, BN, F_IN), lambda i: (0, i, 0)),
        ],
        out_specs=[pl.BlockSpec((BN, 1), lambda i: (i, 0))]
        + [pl.BlockSpec((BN, F_IN), lambda i: (i, 0))] * PERIODS,
        out_shape=[jax.ShapeDtypeStruct((N, 1), _F32)]
        + [jax.ShapeDtypeStruct((N, F_IN), _F32)] * PERIODS,
    )(degp, xt)


def _tc_fold_body(wz, bz, wlz, blz, wh, bh, wlh, blh, az, azb, ah, ahb):
    hi = jax.lax.Precision.HIGHEST
    wlza = wlz[0:F_OUT, :]
    wlha = wlh[0:F_OUT, :]
    az[...] = jnp.dot(wz[...], wlza, precision=hi)
    azb[...] = jnp.dot(bz[...][None, :], wlza, precision=hi)[0] + blz[...]
    ah[...] = jnp.dot(wh[...], wlha, precision=hi)
    ahb[...] = jnp.dot(bh[...][None, :], wlha, precision=hi)[0] + blh[...]


def _tc_fold(wz, bz, wlz, blz, wh, bh, wlh, blh):
    return pl.pallas_call(
        _tc_fold_body,
        out_shape=[
            jax.ShapeDtypeStruct((F_IN, F_OUT), _F32),
            jax.ShapeDtypeStruct((F_OUT,), _F32),
            jax.ShapeDtypeStruct((F_IN, F_OUT), _F32),
            jax.ShapeDtypeStruct((F_OUT,), _F32),
        ],
    )(wz, bz, wlz, blz, wh, bh, wlh, blh)


def _tc_dense_body(att, az, azb, ah, ahb, wo, bo, dinv,
                   y0, y1, y2, y3, s0, s1, s2, s3, out):
    hi = jax.lax.Precision.HIGHEST
    a = att[...]
    e = jnp.exp(a - jnp.max(a))
    probs = e / jnp.sum(e)
    dv = dinv[...]
    azm, ahm = az[...], ah[...]
    azv, ahv = azb[...][None, :], ahb[...][None, :]
    yr = (y0, y1, y2, y3)
    sr = (s0, s1, s2, s3)
    acc = jnp.zeros((BN, F_OUT), _F32)
    for t in range(PERIODS):
        p = dv * (sr[t][...] + yr[t][...])
        z = jax.nn.sigmoid(jnp.dot(p, azm, precision=hi) + azv)
        ht = jnp.tanh(jnp.dot(p, ahm, precision=hi) + ahv)
        acc = acc + probs[t] * ((1.0 - z) * ht)
    out[...] = jnp.dot(jax.nn.relu(acc), wo[...], precision=hi) + bo[...][None, :]


def _tc_dense(att, az, azb, ah, ahb, wo, bo, dinv, ys, ss):
    nf = pl.BlockSpec((BN, F_IN), lambda i: (i, 0))
    return pl.pallas_call(
        _tc_dense_body,
        grid=(N // BN,),
        in_specs=[
            pl.BlockSpec((PERIODS,), lambda i: (0,)),
            pl.BlockSpec((F_IN, F_OUT), lambda i: (0, 0)),
            pl.BlockSpec((F_OUT,), lambda i: (0,)),
            pl.BlockSpec((F_IN, F_OUT), lambda i: (0, 0)),
            pl.BlockSpec((F_OUT,), lambda i: (0,)),
            pl.BlockSpec((F_OUT, PERIODS), lambda i: (0, 0)),
            pl.BlockSpec((PERIODS,), lambda i: (0,)),
            pl.BlockSpec((BN, 1), lambda i: (i, 0)),
        ] + [nf] * (2 * PERIODS),
        out_specs=pl.BlockSpec((BN, PERIODS), lambda i: (i, 0)),
        out_shape=jax.ShapeDtypeStruct((N, PERIODS), _F32),
    )(att, az, azb, ah, ahb, wo, bo, dinv, *ys, *ss)


def kernel(x, edge_index, edge_attributes, attention, Wz, bz, Wlz, blz,
           Wr, br, Wlr, blr, Wh, bh, Wlh, blh, Wo, bo):
    pad = E_PAD - E
    row = jnp.concatenate([edge_index[0], jnp.zeros((pad,), edge_index.dtype)])
    col = jnp.concatenate([edge_index[1], jnp.zeros((pad,), edge_index.dtype)])
    ew = jnp.concatenate([edge_attributes, jnp.zeros((pad,), _F32)])
    rowp = row.reshape(NCHUNKS, CHUNK)
    colp = col.reshape(NCHUNKS, CHUNK)
    ewp = ew.reshape(NCHUNKS, CHUNK)
    xt = jnp.transpose(x, (2, 0, 1))
    z1d = jnp.zeros((640,), _F32)
    z2d = jnp.zeros((640, F_IN), _F32)

    degp = _sc_deg(colp, ewp, z1d)
    degp2 = degp.reshape(NC, N_PAD)[:, :N].T
    dinv, *ys = _tc_pre(degp2, xt)
    az, azb, ah, ahb = _tc_fold(Wz, bz, Wlz, blz, Wh, bh, Wlh, blh)
    ss = _sc_agg(rowp, colp, ewp, z2d, *ys)
    ss = [s.reshape(N_PAD, F_IN)[:N] for s in ss]
    return _tc_dense(attention, az, azb, ah, ahb, Wo, bo, dinv, ys, ss)


# speedup vs baseline: 46.5008x; 1.0020x over previous
import jax
import jax.numpy as jnp
from jax import lax
from jax.experimental import pallas as pl
from jax.experimental.pallas import tpu as pltpu
from jax.experimental.pallas import tpu_sc as plsc

N = 10000
E = 160000
F_IN = 128
F_OUT = 600
PERIODS = 4

CHUNK = 128
E_PAD = 163840
NCHUNKS = E_PAD // CHUNK
NC, NS = 2, 16
BN = 1000

_MESH = plsc.VectorSubcoreMesh(core_axis_name="c", subcore_axis_name="s")
_F32 = jnp.float32


N_PAD = 10240


def _sc_deg_body(colp, ewp, z1d, degp, idxb, payb, acc):
    c = lax.axis_index("c")
    s = lax.axis_index("s")
    pltpu.sync_copy(z1d, acc.at[pl.ds(s * 640, 640)])
    plsc.subcore_barrier()
    w = s * NC + c

    @pl.loop(0, NCHUNKS // (NC * NS))
    def _(j):
        k = w * (NCHUNKS // (NC * NS)) + j
        pltpu.sync_copy(colp.at[pl.ds(k, 1)], idxb)
        pltpu.sync_copy(ewp.at[pl.ds(k, 1)], payb)
        pltpu.sync_copy(payb.at[0], acc.at[idxb.at[0]], add=True)

    plsc.subcore_barrier()
    pltpu.sync_copy(acc.at[pl.ds(s * 640, 640)], degp.at[c, s])


_sc_deg = pl.kernel(
    _sc_deg_body,
    out_type=jax.ShapeDtypeStruct((NC, NS, 640), _F32),
    mesh=_MESH,
    scratch_types=[
        pltpu.VMEM((1, CHUNK), jnp.int32),
        pltpu.VMEM((1, CHUNK), _F32),
        pltpu.VMEM_SHARED((N_PAD,), _F32),
    ],
)


NPT = NCHUNKS // NS
NBUF = 4


def _sc_agg_body(rowp, colp, ewp, z2d, y0, y1, y2, y3,
                 s0, s1, s2, s3, rcb, ewb, gbuf, acc, sem_e, sem_g, sem_s):
    c = lax.axis_index("c")
    s = lax.axis_index("s")

    def period(y, out):
        pltpu.sync_copy(z2d, acc.at[pl.ds(s * 640, 640)])
        plsc.subcore_barrier()
        base = s * NPT

        def gslot(u):
            return gbuf.at[pl.ds((u % 2) * CHUNK, CHUNK)]

        def e_issue(k, u):
            kk = pl.ds(base + k, 1)
            pltpu.async_copy(rowp.at[kk], rcb.at[pl.ds(2 * u, 1)], sem_e)
            pltpu.async_copy(colp.at[kk], rcb.at[pl.ds(2 * u + 1, 1)], sem_e)
            pltpu.async_copy(ewp.at[kk], ewb.at[pl.ds(u, 1)], sem_e)

        def e_wait(u):
            k0 = pl.ds(0, 1)
            pltpu.make_async_copy(rowp.at[k0], rcb.at[pl.ds(2 * u, 1)],
                                  sem_e).wait()
            pltpu.make_async_copy(colp.at[k0], rcb.at[pl.ds(2 * u + 1, 1)],
                                  sem_e).wait()
            pltpu.make_async_copy(ewp.at[k0], ewb.at[pl.ds(u, 1)],
                                  sem_e).wait()

        def g_issue(u):
            pltpu.async_copy(y.at[rcb.at[2 * u, pl.ds(0, 64)]],
                             gslot(u).at[pl.ds(0, 64)], sem_g)
            pltpu.async_copy(y.at[rcb.at[2 * u, pl.ds(64, 64)]],
                             gslot(u).at[pl.ds(64, 64)], sem_g)

        def g_wait(u):
            pltpu.make_async_copy(y.at[pl.ds(0, CHUNK)], gslot(u),
                                  sem_g).wait()

        def sc_issue(u):
            pltpu.async_copy(gslot(u), acc.at[rcb.at[2 * u + 1]], sem_s,
                             add=True)

        def sc_wait(u):
            pltpu.make_async_copy(y.at[pl.ds(0, CHUNK)], gslot(u),
                                  sem_s).wait()

        e_issue(0, 0)
        e_issue(1, 1)
        e_wait(0)
        g_issue(0)

        @pl.loop(0, NPT // NBUF)
        def _(jj):
            for u in range(NBUF):
                j = jj * NBUF + u
                if u == 0:
                    @pl.when(jj > 0)
                    def _():
                        sc_wait(NBUF - 1)
                else:
                    sc_wait(u - 1)

                @pl.when(j + 2 < NPT)
                def _():
                    e_issue(j + 2, (u + 2) % NBUF)

                @pl.when(j + 1 < NPT)
                def _():
                    e_wait((u + 1) % NBUF)
                    g_issue((u + 1) % NBUF)

                g_wait(u)

                @pl.loop(0, CHUNK // 16)
                def _(g):
                    ve = ewb[u, pl.ds(g * 16, 16)]
                    for l in range(16):
                        scl = ve[l]
                        e = (u % 2) * CHUNK + g * 16 + l
                        for q in range(F_IN // 16):
                            slq = pl.ds(q * 16, 16)
                            gbuf[e, slq] = gbuf[e, slq] * scl

                sc_issue(u)

        sc_wait(NBUF - 1)
        plsc.subcore_barrier()
        pltpu.sync_copy(acc.at[pl.ds(s * 640, 640)], out.at[s])
        plsc.subcore_barrier()

    @pl.when(c == 0)
    def _():
        period(y0, s0)
        period(y1, s1)

    @pl.when(c == 1)
    def _():
        period(y2, s2)
        period(y3, s3)


_sc_agg = pl.kernel(
    _sc_agg_body,
    out_type=[jax.ShapeDtypeStruct((NS, 640, F_IN), _F32)] * PERIODS,
    mesh=_MESH,
    scratch_types=[
        pltpu.VMEM((NBUF * 2, CHUNK), jnp.int32),
        pltpu.VMEM((NBUF, CHUNK), _F32),
        pltpu.VMEM((2 * CHUNK, F_IN), _F32),
        pltpu.VMEM_SHARED((N_PAD, F_IN), _F32),
        pltpu.SemaphoreType.DMA,
        pltpu.SemaphoreType.DMA,
        pltpu.SemaphoreType.DMA,
    ],
)


def _tc_pre_body(degp, xt, dinv, y0, y1, y2, y3):
    d = degp[:, 0] + degp[:, 1] + 1.0
    dvc = lax.rsqrt(d)[:, None]
    dinv[...] = dvc
    yrefs = (y0, y1, y2, y3)
    for t in range(PERIODS):
        yrefs[t][...] = xt[t] * dvc


def _tc_pre(degp, xt):
    return pl.pallas_call(
        _tc_pre_body,
        grid=(N // BN,),
        in_specs=[
            pl.BlockSpec((BN, NC), lambda i: (i, 0)),
            pl.BlockSpec((PERIODS, BN, F_IN), lambda i: (0, i, 0)),
        ],
        out_specs=[pl.BlockSpec((BN, 1), lambda i: (i, 0))]
        + [pl.BlockSpec((BN, F_IN), lambda i: (i, 0))] * PERIODS,
        out_shape=[jax.ShapeDtypeStruct((N, 1), _F32)]
        + [jax.ShapeDtypeStruct((N, F_IN), _F32)] * PERIODS,
    )(degp, xt)


def _tc_fold_body(wz, bz, wlz, blz, wh, bh, wlh, blh, az, azb, ah, ahb):
    hi = jax.lax.Precision.HIGHEST
    wlza = wlz[0:F_OUT, :]
    wlha = wlh[0:F_OUT, :]
    az[...] = jnp.dot(wz[...], wlza, precision=hi)
    azb[...] = jnp.dot(bz[...][None, :], wlza, precision=hi)[0] + blz[...]
    ah[...] = jnp.dot(wh[...], wlha, precision=hi)
    ahb[...] = jnp.dot(bh[...][None, :], wlha, precision=hi)[0] + blh[...]


def _tc_fold(wz, bz, wlz, blz, wh, bh, wlh, blh):
    return pl.pallas_call(
        _tc_fold_body,
        out_shape=[
            jax.ShapeDtypeStruct((F_IN, F_OUT), _F32),
            jax.ShapeDtypeStruct((F_OUT,), _F32),
            jax.ShapeDtypeStruct((F_IN, F_OUT), _F32),
            jax.ShapeDtypeStruct((F_OUT,), _F32),
        ],
    )(wz, bz, wlz, blz, wh, bh, wlh, blh)


def _tc_dense_body(att, az, azb, ah, ahb, wo, bo, dinv,
                   y0, y1, y2, y3, s0, s1, s2, s3, out):
    hi = jax.lax.Precision.HIGHEST
    a = att[...]
    e = jnp.exp(a - jnp.max(a))
    probs = e / jnp.sum(e)
    dv = dinv[...]
    azm, ahm = az[...], ah[...]
    azv, ahv = azb[...][None, :], ahb[...][None, :]
    yr = (y0, y1, y2, y3)
    sr = (s0, s1, s2, s3)
    acc = jnp.zeros((BN, F_OUT), _F32)
    for t in range(PERIODS):
        p = dv * (sr[t][...] + yr[t][...])
        z = jax.nn.sigmoid(jnp.dot(p, azm, precision=hi) + azv)
        ht = jnp.tanh(jnp.dot(p, ahm, precision=hi) + ahv)
        acc = acc + probs[t] * ((1.0 - z) * ht)
    out[...] = jnp.dot(jax.nn.relu(acc), wo[...], precision=hi) + bo[...][None, :]


def _tc_dense(att, az, azb, ah, ahb, wo, bo, dinv, ys, ss):
    nf = pl.BlockSpec((BN, F_IN), lambda i: (i, 0))
    return pl.pallas_call(
        _tc_dense_body,
        grid=(N // BN,),
        in_specs=[
            pl.BlockSpec((PERIODS,), lambda i: (0,)),
            pl.BlockSpec((F_IN, F_OUT), lambda i: (0, 0)),
            pl.BlockSpec((F_OUT,), lambda i: (0,)),
            pl.BlockSpec((F_IN, F_OUT), lambda i: (0, 0)),
            pl.BlockSpec((F_OUT,), lambda i: (0,)),
            pl.BlockSpec((F_OUT, PERIODS), lambda i: (0, 0)),
            pl.BlockSpec((PERIODS,), lambda i: (0,)),
            pl.BlockSpec((BN, 1), lambda i: (i, 0)),
        ] + [nf] * (2 * PERIODS),
        out_specs=pl.BlockSpec((BN, PERIODS), lambda i: (i, 0)),
        out_shape=jax.ShapeDtypeStruct((N, PERIODS), _F32),
    )(att, az, azb, ah, ahb, wo, bo, dinv, *ys, *ss)


def kernel(x, edge_index, edge_attributes, attention, Wz, bz, Wlz, blz,
           Wr, br, Wlr, blr, Wh, bh, Wlh, blh, Wo, bo):
    pad = E_PAD - E
    row = jnp.concatenate([edge_index[0], jnp.zeros((pad,), edge_index.dtype)])
    col = jnp.concatenate([edge_index[1], jnp.zeros((pad,), edge_index.dtype)])
    ew = jnp.concatenate([edge_attributes, jnp.zeros((pad,), _F32)])
    rowp = row.reshape(NCHUNKS, CHUNK)
    colp = col.reshape(NCHUNKS, CHUNK)
    ewp = ew.reshape(NCHUNKS, CHUNK)
    xt = jnp.transpose(x, (2, 0, 1))
    z1d = jnp.zeros((640,), _F32)
    z2d = jnp.zeros((640, F_IN), _F32)

    degp = _sc_deg(colp, ewp, z1d)
    degp2 = degp.reshape(NC, N_PAD)[:, :N].T
    dinv, *ys = _tc_pre(degp2, xt)
    az, azb, ah, ahb = _tc_fold(Wz, bz, Wlz, blz, Wh, bh, Wlh, blh)
    ss = _sc_agg(rowp, colp, ewp, z2d, *ys)
    ss = [s.reshape(N_PAD, F_IN)[:N] for s in ss]
    return _tc_dense(attention, az, azb, ah, ahb, Wo, bo, dinv, ys, ss)
